# baseline (speedup 1.0000x reference)
import functools

import jax
import jax.numpy as jnp
from jax import lax
from jax.experimental import pallas as pl
from jax.experimental.pallas import tpu as pltpu
from jax.experimental.pallas import tpu_sc as plsc

NC = 2
NS = 16
NW = NC * NS
K = 128
M0 = 56
M1 = 104
D = 128
BR = 1000


def _sc_mesh():
    return plsc.VectorSubcoreMesh(core_axis_name="c", subcore_axis_name="s")


def _sc_degree(dst2, n_acc):
    rows_per_sub = n_acc // NS
    mx = max(M0, M1)

    @functools.partial(
        pl.kernel,
        out_type=jax.ShapeDtypeStruct((NC, n_acc, 16), jnp.float32),
        mesh=_sc_mesh(),
        scratch_types=[
            pltpu.VMEM((mx, K), jnp.int32),
            pltpu.VMEM((K, 16), jnp.float32),
            pltpu.VMEM((K, 16), jnp.float32),
            pltpu.VMEM_SHARED((n_acc, 16), jnp.float32),
        ],
    )
    def deg_kernel(dst_hbm, out_hbm, idx_v, ones_v, zero_v, acc):
        c = lax.axis_index("c")
        s = lax.axis_index("s")

        @pl.loop(0, K)
        def _(r):
            ones_v[r, :] = jnp.ones((16,), jnp.float32)
            zero_v[r, :] = jnp.zeros((16,), jnp.float32)

        @pl.loop(0, rows_per_sub, step=K)
        def _(t):
            pltpu.sync_copy(zero_v, acc.at[pl.ds(s * rows_per_sub + t, K)])

        plsc.subcore_barrier()

        @pl.when(c == 0)
        def _():
            pltpu.sync_copy(dst_hbm.at[pl.ds(s * M0, M0)],
                            idx_v.at[pl.ds(0, M0)])

            @pl.loop(0, M0)
            def _(j):
                pltpu.sync_copy(ones_v, acc.at[idx_v.at[j]], add=True)

        @pl.when(c == 1)
        def _():
            pltpu.sync_copy(dst_hbm.at[pl.ds(NS * M0 + s * M1, M1)],
                            idx_v.at[pl.ds(0, M1)])

            @pl.loop(0, M1)
            def _(j):
                pltpu.sync_copy(ones_v, acc.at[idx_v.at[j]], add=True)

        plsc.subcore_barrier()
        pltpu.sync_copy(acc.at[pl.ds(s * rows_per_sub, rows_per_sub)],
                        out_hbm.at[c, pl.ds(s * rows_per_sub, rows_per_sub)])

    return deg_kernel(dst2)


def _sc_aggregate(u, src2, dst2, n_acc):
    rows_per_sub = n_acc // NS
    mx = max(M0, M1)

    @functools.partial(
        pl.kernel,
        out_type=jax.ShapeDtypeStruct((NC, n_acc, D), jnp.float32),
        mesh=_sc_mesh(),
        scratch_types=[
            pltpu.VMEM((mx, K), jnp.int32),
            pltpu.VMEM((mx, K), jnp.int32),
            pltpu.VMEM((K, D), jnp.float32),
            pltpu.VMEM_SHARED((n_acc, D), jnp.float32),
        ],
    )
    def agg_kernel(u_hbm, src_hbm, dst_hbm, out_hbm,
                   src_v, dst_v, bufa, acc):
        c = lax.axis_index("c")
        s = lax.axis_index("s")

        @pl.loop(0, 64)
        def _(r):
            @pl.loop(0, D, step=16)
            def _(cc):
                bufa[r, pl.ds(cc, 16)] = jnp.zeros((16,), jnp.float32)

        @pl.loop(0, rows_per_sub, step=64)
        def _(t):
            pltpu.sync_copy(bufa.at[pl.ds(0, 64)],
                            acc.at[pl.ds(s * rows_per_sub + t, 64)])

        plsc.subcore_barrier()

        @pl.when(c == 0)
        def _():
            pltpu.sync_copy(src_hbm.at[pl.ds(s * M0, M0)],
                            src_v.at[pl.ds(0, M0)])
            pltpu.sync_copy(dst_hbm.at[pl.ds(s * M0, M0)],
                            dst_v.at[pl.ds(0, M0)])

            @pl.loop(0, M0)
            def _(j):
                pltpu.sync_copy(u_hbm.at[src_v.at[j]], bufa)
                pltpu.sync_copy(bufa, acc.at[dst_v.at[j]], add=True)

        @pl.when(c == 1)
        def _():
            pltpu.sync_copy(src_hbm.at[pl.ds(NS * M0 + s * M1, M1)],
                            src_v.at[pl.ds(0, M1)])
            pltpu.sync_copy(dst_hbm.at[pl.ds(NS * M0 + s * M1, M1)],
                            dst_v.at[pl.ds(0, M1)])

            @pl.loop(0, M1)
            def _(j):
                pltpu.sync_copy(u_hbm.at[src_v.at[j]], bufa)
                pltpu.sync_copy(bufa, acc.at[dst_v.at[j]], add=True)

        plsc.subcore_barrier()
        pltpu.sync_copy(acc.at[pl.ds(s * rows_per_sub, rows_per_sub)],
                        out_hbm.at[c, pl.ds(s * rows_per_sub, rows_per_sub)])

    return agg_kernel(u, src2, dst2)


def _tc_matmul(x, w):
    n = x.shape[0]

    def body(x_ref, w_ref, o_ref):
        o_ref[...] = jnp.dot(x_ref[...], w_ref[...],
                             preferred_element_type=jnp.float32)

    return pl.pallas_call(
        body,
        grid=(n // BR,),
        in_specs=[pl.BlockSpec((BR, D), lambda i: (i, 0)),
                  pl.BlockSpec((D, D), lambda i: (0, 0))],
        out_specs=pl.BlockSpec((BR, D), lambda i: (i, 0)),
        out_shape=jax.ShapeDtypeStruct((n, D), jnp.float32),
    )(x, w)


def _tc_scale(degp, h):
    n = h.shape[0]

    def body(deg_ref, h_ref, dis_ref, u_ref):
        deg = deg_ref[0] + deg_ref[1]
        total = deg[:, 0:1] + 1.0
        dis = lax.rsqrt(total)
        dis_ref[...] = dis
        u_ref[...] = dis * h_ref[...]

    return pl.pallas_call(
        body,
        grid=(n // BR,),
        in_specs=[pl.BlockSpec((NC, BR, 16), lambda i: (0, i, 0)),
                  pl.BlockSpec((BR, D), lambda i: (i, 0))],
        out_specs=[pl.BlockSpec((BR, 1), lambda i: (i, 0)),
                   pl.BlockSpec((BR, D), lambda i: (i, 0))],
        out_shape=[jax.ShapeDtypeStruct((n, 1), jnp.float32),
                   jax.ShapeDtypeStruct((n, D), jnp.float32)],
    )(degp, h)


def _tc_combine_mid(aggp, u1, dis, b1, w2):
    n = u1.shape[0]

    def body(agg_ref, u1_ref, dis_ref, b1_ref, w2_ref, u2_ref):
        agg = agg_ref[0] + agg_ref[1]
        dis = dis_ref[...]
        l1 = jnp.maximum(dis * (agg + u1_ref[...]) + b1_ref[...], 0.0)
        h2 = jnp.dot(l1, w2_ref[...], preferred_element_type=jnp.float32)
        u2_ref[...] = dis * h2

    return pl.pallas_call(
        body,
        grid=(n // BR,),
        in_specs=[pl.BlockSpec((NC, BR, D), lambda i: (0, i, 0)),
                  pl.BlockSpec((BR, D), lambda i: (i, 0)),
                  pl.BlockSpec((BR, 1), lambda i: (i, 0)),
                  pl.BlockSpec((1, D), lambda i: (0, 0)),
                  pl.BlockSpec((D, D), lambda i: (0, 0))],
        out_specs=pl.BlockSpec((BR, D), lambda i: (i, 0)),
        out_shape=jax.ShapeDtypeStruct((n, D), jnp.float32),
    )(aggp, u1, dis, b1, w2)


def _tc_combine_out(aggp, u2, dis, b2):
    n = u2.shape[0]

    def body(agg_ref, u2_ref, dis_ref, b2_ref, z_ref):
        agg = agg_ref[0] + agg_ref[1]
        z_ref[...] = dis_ref[...] * (agg + u2_ref[...]) + b2_ref[...]

    return pl.pallas_call(
        body,
        grid=(n // BR,),
        in_specs=[pl.BlockSpec((NC, BR, D), lambda i: (0, i, 0)),
                  pl.BlockSpec((BR, D), lambda i: (i, 0)),
                  pl.BlockSpec((BR, 1), lambda i: (i, 0)),
                  pl.BlockSpec((1, D), lambda i: (0, 0))],
        out_specs=pl.BlockSpec((BR, D), lambda i: (i, 0)),
        out_shape=jax.ShapeDtypeStruct((n, D), jnp.float32),
    )(aggp, u2, dis, b2)


def kernel(x, edge_index, W1, b1, W2, b2):
    n = x.shape[0]
    e = edge_index.shape[1]

    tot_chunks = NS * (M0 + M1)
    e_pad = tot_chunks * K
    assert e_pad >= e
    rows_per_sub = -(-(n + 1) // (NS * K)) * K
    n_acc = rows_per_sub * NS

    src = edge_index[0].astype(jnp.int32)
    dst = edge_index[1].astype(jnp.int32)
    pad = e_pad - e
    src2 = jnp.concatenate([src, jnp.zeros((pad,), jnp.int32)]
                           ).reshape(tot_chunks, K)
    dst2 = jnp.concatenate([dst, jnp.full((pad,), n, jnp.int32)]
                           ).reshape(tot_chunks, K)

    degp = _sc_degree(dst2, n_acc)
    h1 = _tc_matmul(x, W1)
    dis, u1 = _tc_scale(degp, h1)

    aggp1 = _sc_aggregate(u1, src2, dst2, n_acc)
    u2 = _tc_combine_mid(aggp1, u1, dis, b1.reshape(1, D), W2)

    aggp2 = _sc_aggregate(u2, src2, dst2, n_acc)
    z = _tc_combine_out(aggp2, u2, dis, b2.reshape(1, D))
    return z


# speedup vs baseline: 1.7019x; 1.7019x over previous
import functools

import jax
import jax.numpy as jnp
from jax import lax
from jax.experimental import pallas as pl
from jax.experimental.pallas import tpu as pltpu
from jax.experimental.pallas import tpu_sc as plsc

NC = 2
NS = 16
NW = NC * NS
K = 128
D = 128
BR = 1000


def _sc_mesh():
    return plsc.VectorSubcoreMesh(core_axis_name="c", subcore_axis_name="s")


def _sc_degree(dst3, n_acc):
    n_chunks = dst3.shape[1]
    rows_per_sub = n_acc // NS

    @functools.partial(
        pl.kernel,
        out_type=jax.ShapeDtypeStruct((NC, n_acc, 16), jnp.float32),
        mesh=_sc_mesh(),
        scratch_types=[
            pltpu.VMEM((n_chunks, K), jnp.int32),
            pltpu.VMEM((K, 16), jnp.float32),
            pltpu.VMEM((K, 16), jnp.float32),
            pltpu.VMEM_SHARED((n_acc, 16), jnp.float32),
        ],
    )
    def deg_kernel(dst_hbm, out_hbm, idx_v, ones_v, zero_v, acc):
        c = lax.axis_index("c")
        s = lax.axis_index("s")
        w = c * NS + s

        @pl.loop(0, K)
        def _(r):
            ones_v[r, :] = jnp.ones((16,), jnp.float32)
            zero_v[r, :] = jnp.zeros((16,), jnp.float32)

        @pl.loop(0, rows_per_sub, step=K)
        def _(t):
            pltpu.sync_copy(zero_v, acc.at[pl.ds(s * rows_per_sub + t, K)])

        plsc.subcore_barrier()
        pltpu.sync_copy(dst_hbm.at[w], idx_v)

        @pl.loop(0, n_chunks)
        def _(j):
            pltpu.sync_copy(ones_v, acc.at[idx_v.at[j]], add=True)

        plsc.subcore_barrier()
        pltpu.sync_copy(acc.at[pl.ds(s * rows_per_sub, rows_per_sub)],
                        out_hbm.at[c, pl.ds(s * rows_per_sub, rows_per_sub)])

    return deg_kernel(dst3)


def _sc_aggregate(u, src3, dst3, n_acc):
    n_chunks = src3.shape[1]
    rows_per_sub = n_acc // NS

    @functools.partial(
        pl.kernel,
        out_type=jax.ShapeDtypeStruct((NC, n_acc, D), jnp.float32),
        mesh=_sc_mesh(),
        scratch_types=[
            pltpu.VMEM((n_chunks, K), jnp.int32),
            pltpu.VMEM((n_chunks, K), jnp.int32),
            pltpu.VMEM((K, D), jnp.float32),
            pltpu.VMEM_SHARED((n_acc, D), jnp.float32),
        ],
    )
    def agg_kernel(u_hbm, src_hbm, dst_hbm, out_hbm,
                   src_v, dst_v, bufa, acc):
        c = lax.axis_index("c")
        s = lax.axis_index("s")
        w = c * NS + s

        @pl.loop(0, K)
        def _(r):
            @pl.loop(0, D, step=16)
            def _(cc):
                bufa[r, pl.ds(cc, 16)] = jnp.zeros((16,), jnp.float32)

        @pl.loop(0, rows_per_sub, step=K)
        def _(t):
            pltpu.sync_copy(bufa, acc.at[pl.ds(s * rows_per_sub + t, K)])

        plsc.subcore_barrier()
        pltpu.sync_copy(src_hbm.at[w], src_v)
        pltpu.sync_copy(dst_hbm.at[w], dst_v)

        @pl.loop(0, n_chunks)
        def _(j):
            pltpu.sync_copy(u_hbm.at[src_v.at[j]], bufa)
            pltpu.sync_copy(bufa, acc.at[dst_v.at[j]], add=True)

        plsc.subcore_barrier()
        pltpu.sync_copy(acc.at[pl.ds(s * rows_per_sub, rows_per_sub)],
                        out_hbm.at[c, pl.ds(s * rows_per_sub, rows_per_sub)])

    return agg_kernel(u, src3, dst3)


def _tc_matmul(x, w):
    n = x.shape[0]

    def body(x_ref, w_ref, o_ref):
        o_ref[...] = jnp.dot(x_ref[...], w_ref[...],
                             preferred_element_type=jnp.float32)

    return pl.pallas_call(
        body,
        grid=(n // BR,),
        in_specs=[pl.BlockSpec((BR, D), lambda i: (i, 0)),
                  pl.BlockSpec((D, D), lambda i: (0, 0))],
        out_specs=pl.BlockSpec((BR, D), lambda i: (i, 0)),
        out_shape=jax.ShapeDtypeStruct((n, D), jnp.float32),
    )(x, w)


def _tc_scale(degp, h):
    n = h.shape[0]

    def body(deg_ref, h_ref, dis_ref, u_ref):
        deg = deg_ref[0] + deg_ref[1]
        total = deg[:, 0:1] + 1.0
        dis = lax.rsqrt(total)
        dis_ref[...] = dis
        u_ref[...] = dis * h_ref[...]

    return pl.pallas_call(
        body,
        grid=(n // BR,),
        in_specs=[pl.BlockSpec((NC, BR, 16), lambda i: (0, i, 0)),
                  pl.BlockSpec((BR, D), lambda i: (i, 0))],
        out_specs=[pl.BlockSpec((BR, 1), lambda i: (i, 0)),
                   pl.BlockSpec((BR, D), lambda i: (i, 0))],
        out_shape=[jax.ShapeDtypeStruct((n, 1), jnp.float32),
                   jax.ShapeDtypeStruct((n, D), jnp.float32)],
    )(degp, h)


def _tc_combine_mid(aggp, u1, dis, b1, w2):
    n = u1.shape[0]

    def body(agg_ref, u1_ref, dis_ref, b1_ref, w2_ref, u2_ref):
        agg = agg_ref[0] + agg_ref[1]
        dis = dis_ref[...]
        l1 = jnp.maximum(dis * (agg + u1_ref[...]) + b1_ref[...], 0.0)
        h2 = jnp.dot(l1, w2_ref[...], preferred_element_type=jnp.float32)
        u2_ref[...] = dis * h2

    return pl.pallas_call(
        body,
        grid=(n // BR,),
        in_specs=[pl.BlockSpec((NC, BR, D), lambda i: (0, i, 0)),
                  pl.BlockSpec((BR, D), lambda i: (i, 0)),
                  pl.BlockSpec((BR, 1), lambda i: (i, 0)),
                  pl.BlockSpec((1, D), lambda i: (0, 0)),
                  pl.BlockSpec((D, D), lambda i: (0, 0))],
        out_specs=pl.BlockSpec((BR, D), lambda i: (i, 0)),
        out_shape=jax.ShapeDtypeStruct((n, D), jnp.float32),
    )(aggp, u1, dis, b1, w2)


def _tc_combine_out(aggp, u2, dis, b2):
    n = u2.shape[0]

    def body(agg_ref, u2_ref, dis_ref, b2_ref, z_ref):
        agg = agg_ref[0] + agg_ref[1]
        z_ref[...] = dis_ref[...] * (agg + u2_ref[...]) + b2_ref[...]

    return pl.pallas_call(
        body,
        grid=(n // BR,),
        in_specs=[pl.BlockSpec((NC, BR, D), lambda i: (0, i, 0)),
                  pl.BlockSpec((BR, D), lambda i: (i, 0)),
                  pl.BlockSpec((BR, 1), lambda i: (i, 0)),
                  pl.BlockSpec((1, D), lambda i: (0, 0))],
        out_specs=pl.BlockSpec((BR, D), lambda i: (i, 0)),
        out_shape=jax.ShapeDtypeStruct((n, D), jnp.float32),
    )(aggp, u2, dis, b2)


def kernel(x, edge_index, W1, b1, W2, b2):
    n = x.shape[0]
    e = edge_index.shape[1]

    per_w = -(-e // (NW * K)) * K
    e_pad = per_w * NW
    n_chunks = per_w // K
    rows_per_sub = -(-(n + 1) // (NS * K)) * K
    n_acc = rows_per_sub * NS

    src = edge_index[0].astype(jnp.int32)
    dst = edge_index[1].astype(jnp.int32)
    pad = e_pad - e
    src3 = jnp.concatenate([src, jnp.zeros((pad,), jnp.int32)]
                           ).reshape(NW, n_chunks, K)
    dst3 = jnp.concatenate([dst, jnp.full((pad,), n, jnp.int32)]
                           ).reshape(NW, n_chunks, K)

    degp = _sc_degree(dst3, n_acc)
    h1 = _tc_matmul(x, W1)
    dis, u1 = _tc_scale(degp, h1)

    aggp1 = _sc_aggregate(u1, src3, dst3, n_acc)
    u2 = _tc_combine_mid(aggp1, u1, dis, b1.reshape(1, D), W2)

    aggp2 = _sc_aggregate(u2, src3, dst3, n_acc)
    z = _tc_combine_out(aggp2, u2, dis, b2.reshape(1, D))
    return z
